# baseline (device time: 23370 ns/iter reference)
import jax
import jax.numpy as jnp
from jax import lax
from jax.experimental import pallas as pl
from jax.experimental.pallas import tpu as pltpu

N_DEV = 8
P = 4
N_A = 3
N_B = 5
N_STREAMS = N_A + N_B


def kernel(x, dy):
    m, d_in = x.shape
    _, f = dy.shape
    rows = d_in // N_DEV
    fq = f // N_STREAMS

    def body(x_ref, dy_ref, out_ref, acc_ref,
             abuf, arecv, asum, azs, azr,
             zbs, zbr, pair, bbuf, brecv,
             a_s, a_r, az_s, az_r, zb_s, zb_r, b_s, b_r):
        my = lax.axis_index("i")
        r = lax.rem(my, P)
        z = lax.div(my, P)
        left = z * P + lax.rem(r + P - 1, P)
        right = z * P + lax.rem(r + 1, P)
        diag = z * P + lax.rem(r + 2, P)
        zpartner = lax.rem(my + P, N_DEV)

        barrier_sem = pltpu.get_barrier_semaphore()
        for nbr in (left, right, diag, zpartner):
            pl.semaphore_signal(
                barrier_sem, inc=1,
                device_id=(nbr,), device_id_type=pl.DeviceIdType.MESH,
            )

        def gemm_block(st):
            acc_ref[:, pl.ds(st * fq, fq)] = lax.dot_general(
                x_ref[:, :].astype(jnp.bfloat16),
                dy_ref[:, pl.ds(st * fq, fq)].astype(jnp.bfloat16),
                dimension_numbers=(((0,), (0,)), ((), ())),
                preferred_element_type=jnp.float32,
            )

        def cols(st):
            return pl.ds(st * fq, fq)

        def inplane_dev(o):
            return z * P + lax.rem(r + o, P)

        rdmas = []

        zb_rdmas = {}
        for bi in range(N_B):
            st = N_A + bi
            gemm_block(st)
            if bi == 0:
                pl.semaphore_wait(barrier_sem, 4)
            zbs[bi, :, :] = acc_ref[
                pl.ds((1 - z) * P * rows, P * rows), cols(st)
            ].astype(jnp.bfloat16)
            c = pltpu.make_async_remote_copy(
                src_ref=zbs.at[bi], dst_ref=zbr.at[bi],
                send_sem=zb_s.at[bi], recv_sem=zb_r.at[bi],
                device_id=(zpartner,), device_id_type=pl.DeviceIdType.MESH,
            )
            c.start()
            zb_rdmas[bi] = c
            rdmas.append(c)

        a_rdmas = {}
        for ai in range(N_A):
            st = ai
            gemm_block(st)
            for o in (1, 2, 3):
                c4 = lax.rem(r + o, P)
                abuf[ai, o - 1, 0, :, :] = acc_ref[
                    pl.ds((c4 + P * z) * rows, rows), cols(st)
                ].astype(jnp.bfloat16)
                abuf[ai, o - 1, 1, :, :] = acc_ref[
                    pl.ds((c4 + P * (1 - z)) * rows, rows), cols(st)
                ].astype(jnp.bfloat16)
                c = pltpu.make_async_remote_copy(
                    src_ref=abuf.at[ai, o - 1], dst_ref=arecv.at[ai, o - 1],
                    send_sem=a_s.at[ai, o - 1], recv_sem=a_r.at[ai, o - 1],
                    device_id=(inplane_dev(o),),
                    device_id_type=pl.DeviceIdType.MESH,
                )
                c.start()
                a_rdmas[(ai, o)] = c
                rdmas.append(c)

        az_rdmas = {}
        for ai in range(N_A):
            st = ai
            for o in (1, 2, 3):
                a_rdmas[(ai, o)].wait_recv()
            asum[ai, :, :] = (
                acc_ref[pl.ds((r + P * z) * rows, rows), cols(st)]
                + arecv[ai, 0, 0, :, :].astype(jnp.float32)
                + arecv[ai, 1, 0, :, :].astype(jnp.float32)
                + arecv[ai, 2, 0, :, :].astype(jnp.float32)
            )
            azs[ai, :, :] = (
                acc_ref[pl.ds((r + P * (1 - z)) * rows, rows), cols(st)]
                + arecv[ai, 0, 1, :, :].astype(jnp.float32)
                + arecv[ai, 1, 1, :, :].astype(jnp.float32)
                + arecv[ai, 2, 1, :, :].astype(jnp.float32)
            ).astype(jnp.bfloat16)
            c = pltpu.make_async_remote_copy(
                src_ref=azs.at[ai], dst_ref=azr.at[ai],
                send_sem=az_s.at[ai], recv_sem=az_r.at[ai],
                device_id=(zpartner,), device_id_type=pl.DeviceIdType.MESH,
            )
            c.start()
            az_rdmas[ai] = c
            rdmas.append(c)

        b_rdmas = {}
        for bi in range(N_B):
            st = N_A + bi
            zb_rdmas[bi].wait_recv()
            pair[bi, :, :] = (
                zbr[bi, :, :].astype(jnp.float32)
                + acc_ref[pl.ds(z * P * rows, P * rows), cols(st)]
            )
            for o in (1, 2, 3):
                c4 = lax.rem(r + o, P)
                bbuf[bi, o - 1, :, :] = pair[
                    bi, pl.ds(c4 * rows, rows), :
                ].astype(jnp.bfloat16)
                c = pltpu.make_async_remote_copy(
                    src_ref=bbuf.at[bi, o - 1], dst_ref=brecv.at[bi, o - 1],
                    send_sem=b_s.at[bi, o - 1], recv_sem=b_r.at[bi, o - 1],
                    device_id=(inplane_dev(o),),
                    device_id_type=pl.DeviceIdType.MESH,
                )
                c.start()
                b_rdmas[(bi, o)] = c
                rdmas.append(c)

        for ai in range(N_A):
            az_rdmas[ai].wait_recv()
            out_ref[:, cols(ai)] = (
                asum[ai, :, :] + azr[ai, :, :].astype(jnp.float32)
            )

        for bi in range(N_B):
            st = N_A + bi
            for o in (1, 2, 3):
                b_rdmas[(bi, o)].wait_recv()
            out_ref[:, cols(st)] = (
                pair[bi, pl.ds(r * rows, rows), :]
                + brecv[bi, 0, :, :].astype(jnp.float32)
                + brecv[bi, 1, :, :].astype(jnp.float32)
                + brecv[bi, 2, :, :].astype(jnp.float32)
            )

        for c in rdmas:
            c.wait_send()

    return pl.pallas_call(
        body,
        out_shape=jax.ShapeDtypeStruct((rows, f), jnp.float32),
        in_specs=[
            pl.BlockSpec(memory_space=pltpu.VMEM),
            pl.BlockSpec(memory_space=pltpu.VMEM),
        ],
        out_specs=pl.BlockSpec(memory_space=pltpu.VMEM),
        scratch_shapes=[
            pltpu.VMEM((d_in, f), jnp.float32),
            pltpu.VMEM((N_A, 3, 2, rows, fq), jnp.bfloat16),
            pltpu.VMEM((N_A, 3, 2, rows, fq), jnp.bfloat16),
            pltpu.VMEM((N_A, rows, fq), jnp.float32),
            pltpu.VMEM((N_A, rows, fq), jnp.bfloat16),
            pltpu.VMEM((N_A, rows, fq), jnp.bfloat16),
            pltpu.VMEM((N_B, P * rows, fq), jnp.bfloat16),
            pltpu.VMEM((N_B, P * rows, fq), jnp.bfloat16),
            pltpu.VMEM((N_B, P * rows, fq), jnp.float32),
            pltpu.VMEM((N_B, 3, rows, fq), jnp.bfloat16),
            pltpu.VMEM((N_B, 3, rows, fq), jnp.bfloat16),
            pltpu.SemaphoreType.DMA((N_A, 3)),
            pltpu.SemaphoreType.DMA((N_A, 3)),
            pltpu.SemaphoreType.DMA((N_A,)),
            pltpu.SemaphoreType.DMA((N_A,)),
            pltpu.SemaphoreType.DMA((N_B,)),
            pltpu.SemaphoreType.DMA((N_B,)),
            pltpu.SemaphoreType.DMA((N_B, 3)),
            pltpu.SemaphoreType.DMA((N_B, 3)),
        ],
        compiler_params=pltpu.CompilerParams(collective_id=0),
    )(x, dy)


# device time: 21822 ns/iter; 1.0709x vs baseline; 1.0709x over previous
import jax
import jax.numpy as jnp
from jax import lax
from jax.experimental import pallas as pl
from jax.experimental.pallas import tpu as pltpu

N_DEV = 8
P = 4
IP_HOPS = P - 1
S = 4


def kernel(x, dy):
    m, d_in = x.shape
    _, f = dy.shape
    rows = d_in // N_DEV
    n_streams = 2 * S
    fq = f // n_streams

    stream_dirs = [k % 2 == 0 for k in range(n_streams)]

    def body(x_ref, dy_ref, out_ref, acc_ref, *rest):
        n = n_streams
        comm = rest[0:n]
        zrecv = rest[n:2 * n]
        ip_send = rest[2 * n:3 * n]
        ip_recv = rest[3 * n:4 * n]
        z_send = rest[4 * n:5 * n]
        z_recv = rest[5 * n:6 * n]

        my = lax.axis_index("i")
        r = lax.rem(my, P)
        z = lax.div(my, P)
        left = z * P + lax.rem(r + P - 1, P)
        right = z * P + lax.rem(r + 1, P)
        zpartner = lax.rem(my + P, N_DEV)

        barrier_sem = pltpu.get_barrier_semaphore()
        for nbr in (left, right, zpartner):
            pl.semaphore_signal(
                barrier_sem, inc=1,
                device_id=(nbr,), device_id_type=pl.DeviceIdType.MESH,
            )

        def group_at(st, s):
            if stream_dirs[st]:
                return lax.rem(r + P - 1 - s, P)
            return lax.rem(r + 1 + s, P)

        def acc_comp(g, comp_is_mine, st):
            zz = z if comp_is_mine else (1 - z)
            return acc_ref[pl.ds((g + P * zz) * rows, rows),
                           pl.ds(st * fq, fq)]

        def make_ip(st, s):
            return pltpu.make_async_remote_copy(
                src_ref=comm[st].at[s],
                dst_ref=comm[st].at[s + 1],
                send_sem=ip_send[st].at[s],
                recv_sem=ip_recv[st].at[s],
                device_id=(right if stream_dirs[st] else left,),
                device_id_type=pl.DeviceIdType.MESH,
            )

        rdmas = {}
        zdmas = {}

        for st in range(n_streams):
            acc_ref[:, pl.ds(st * fq, fq)] = lax.dot_general(
                x_ref[:, :].astype(jnp.bfloat16),
                dy_ref[:, pl.ds(st * fq, fq)].astype(jnp.bfloat16),
                dimension_numbers=(((0,), (0,)), ((), ())),
                preferred_element_type=jnp.float32,
            )
            g0 = group_at(st, 0)
            comm[st][0, 0, :, :] = acc_comp(g0, True, st).astype(jnp.bfloat16)
            comm[st][0, 1, :, :] = acc_comp(g0, False, st).astype(jnp.bfloat16)
            if st == 0:
                pl.semaphore_wait(barrier_sem, 3)
            rdmas[(st, 0)] = make_ip(st, 0)
            rdmas[(st, 0)].start()

        for s in range(1, IP_HOPS):
            for st in range(n_streams):
                rdmas[(st, s - 1)].wait_recv()
                g = group_at(st, s)
                comm[st][s, 0, :, :] = (
                    comm[st][s, 0, :, :].astype(jnp.float32)
                    + acc_comp(g, True, st)
                ).astype(jnp.bfloat16)
                comm[st][s, 1, :, :] = (
                    comm[st][s, 1, :, :].astype(jnp.float32)
                    + acc_comp(g, False, st)
                ).astype(jnp.bfloat16)
                rdmas[(st, s)] = make_ip(st, s)
                rdmas[(st, s)].start()

        for st in range(n_streams):
            rdmas[(st, IP_HOPS - 1)].wait_recv()
            comm[st][IP_HOPS, 0, :, :] = (
                comm[st][IP_HOPS, 0, :, :].astype(jnp.float32)
                + acc_comp(r, True, st)
            ).astype(jnp.bfloat16)
            comm[st][IP_HOPS, 1, :, :] = (
                comm[st][IP_HOPS, 1, :, :].astype(jnp.float32)
                + acc_comp(r, False, st)
            ).astype(jnp.bfloat16)
            zdmas[st] = pltpu.make_async_remote_copy(
                src_ref=comm[st].at[IP_HOPS, 1],
                dst_ref=zrecv[st],
                send_sem=z_send[st],
                recv_sem=z_recv[st],
                device_id=(zpartner,),
                device_id_type=pl.DeviceIdType.MESH,
            )
            zdmas[st].start()

        for st in range(n_streams):
            zdmas[st].wait_recv()
            out_ref[:, pl.ds(st * fq, fq)] = (
                comm[st][IP_HOPS, 0, :, :].astype(jnp.float32)
                + zrecv[st][:, :].astype(jnp.float32)
            )

        for st in range(n_streams):
            for s in range(IP_HOPS):
                rdmas[(st, s)].wait_send()
            zdmas[st].wait_send()

    return pl.pallas_call(
        body,
        out_shape=jax.ShapeDtypeStruct((rows, f), jnp.float32),
        in_specs=[
            pl.BlockSpec(memory_space=pltpu.VMEM),
            pl.BlockSpec(memory_space=pltpu.VMEM),
        ],
        out_specs=pl.BlockSpec(memory_space=pltpu.VMEM),
        scratch_shapes=(
            [pltpu.VMEM((d_in, f), jnp.float32)]
            + [pltpu.VMEM((IP_HOPS + 1, 2, rows, fq), jnp.bfloat16)
               for _ in range(n_streams)]
            + [pltpu.VMEM((rows, fq), jnp.bfloat16)
               for _ in range(n_streams)]
            + [pltpu.SemaphoreType.DMA((IP_HOPS,))
               for _ in range(n_streams)]
            + [pltpu.SemaphoreType.DMA((IP_HOPS,))
               for _ in range(n_streams)]
            + [pltpu.SemaphoreType.DMA for _ in range(n_streams)]
            + [pltpu.SemaphoreType.DMA for _ in range(n_streams)]
        ),
        compiler_params=pltpu.CompilerParams(collective_id=0),
    )(x, dy)
